# baseline (speedup 1.0000x reference)
import jax
import jax.numpy as jnp
from jax import lax
from jax.experimental import pallas as pl

N_K = 2048
D = 1024
THRESHOLD = 0.1
BLK = 256
N_BLKS = N_K // BLK
B = 1024


def _prep_body(x_ref, e_ref, y_ref, a_ref):
    v = jnp.dot(x_ref[...], e_ref[...], preferred_element_type=jnp.float32)
    mu = jnp.mean(v, axis=-1, keepdims=True)
    var = jnp.mean((v - mu) ** 2, axis=-1, keepdims=True)
    ln = (v - mu) * jax.lax.rsqrt(var + 1e-5)
    y = jnp.maximum(ln, 0.0)
    y_ref[...] = y
    x0 = y[0:1, :]
    a_ref[...] = jnp.where(x0 > THRESHOLD, x0, 0.0)


def _mega1_body(dx_ref, a_row_ref, a_col_ref, idx_col_ref, gcn_ref, vb_ref):
    dx = dx_ref[...].astype(jnp.bfloat16)
    a_row = a_row_ref[...]
    a_col = a_col_ref[...].astype(jnp.bfloat16)
    idx_col = idx_col_ref[...]

    cols_n = jax.lax.broadcasted_iota(jnp.int32, (B, N_K), 1)
    sel = idx_col == cols_n
    cb = sel.astype(jnp.bfloat16)

    dxc = jnp.dot(cb, dx, preferred_element_type=jnp.float32) \
        .astype(jnp.bfloat16)
    ac_col = jnp.dot(cb, a_col, preferred_element_type=jnp.float32)

    gate = jax.nn.sigmoid(jax.lax.dot_general(
        dxc, dx, (((1,), (1,)), ((), ())),
        preferred_element_type=jnp.float32))
    offd_n = (idx_col != cols_n).astype(jnp.float32)
    gcn_f = gate * (ac_col * a_row) * offd_n * (1.0 / N_K)
    gcn = gcn_f.astype(jnp.bfloat16)
    gcn_ref[...] = gcn

    gate_c = jax.nn.sigmoid(jax.lax.dot_general(
        dxc, dxc, (((1,), (1,)), ((), ())),
        preferred_element_type=jnp.float32))
    ii = jax.lax.broadcasted_iota(jnp.int32, (B, B), 0)
    jj = jax.lax.broadcasted_iota(jnp.int32, (B, B), 1)
    ac_row = ac_col.reshape(1, B)
    gc_f = gate_c * (ac_col * ac_row) * (ii != jj).astype(jnp.float32) \
        * (1.0 / N_K)
    gc = gc_f.astype(jnp.bfloat16)

    g2_f = jnp.dot(gc, gc, preferred_element_type=jnp.float32)
    g2 = g2_f.astype(jnp.bfloat16)
    mm = jnp.dot(gc, g2, preferred_element_type=jnp.float32)
    eye_b = jnp.where(ii == jj, 1.0, 0.0)
    t3 = (eye_b + gc_f + g2_f + mm).astype(jnp.bfloat16)

    v = jnp.dot(t3, gcn, preferred_element_type=jnp.float32)
    vb_ref[...] = (v + sel.astype(jnp.float32)).astype(jnp.bfloat16)


def _mega2_body(gcn_blk_ref, v_full_ref, o_ref):
    i = pl.program_id(0)
    f = jax.lax.dot_general(
        gcn_blk_ref[...], v_full_ref[...], (((0,), (0,)), ((), ())),
        preferred_element_type=jnp.float32)
    rows = jax.lax.broadcasted_iota(jnp.int32, (BLK, N_K), 0) + i * BLK
    cols = jax.lax.broadcasted_iota(jnp.int32, (BLK, N_K), 1)
    o_ref[...] = f + (rows == cols).astype(jnp.float32)


def _g_body(dx_blk_ref, dx_full_ref, a_blk_ref, a_full_ref, g_ref):
    i = pl.program_id(0)
    dots = jax.lax.dot_general(
        dx_blk_ref[...], dx_full_ref[...], (((1,), (1,)), ((), ())),
        preferred_element_type=jnp.float32)
    gate = jax.nn.sigmoid(dots)
    a_col = a_blk_ref[...].reshape(BLK, 1)
    a_row = a_full_ref[...]
    rows = jax.lax.broadcasted_iota(jnp.int32, (BLK, N_K), 0) + i * BLK
    cols = jax.lax.broadcasted_iota(jnp.int32, (BLK, N_K), 1)
    offdiag = (rows != cols).astype(jnp.float32)
    g = gate * (a_col * a_row) * offdiag * (1.0 / N_K)
    g_ref[...] = g.astype(jnp.bfloat16)


def _mm_body(a_blk_ref, b_full_ref, o_ref):
    o_ref[...] = jnp.dot(
        a_blk_ref[...], b_full_ref[...], preferred_element_type=jnp.float32
    ).astype(jnp.bfloat16)


def _mm_p_body(g_blk_ref, g2_full_ref, g3_ref, p_ref):
    i = pl.program_id(0)
    g_blk = g_blk_ref[...]
    g2_full = g2_full_ref[...]
    g3_ref[...] = jnp.dot(
        g_blk, g2_full, preferred_element_type=jnp.float32
    ).astype(jnp.bfloat16)
    g2_blk = g2_full_ref[pl.ds(i * BLK, BLK), :]
    rows = jax.lax.broadcasted_iota(jnp.int32, (BLK, N_K), 0) + i * BLK
    cols = jax.lax.broadcasted_iota(jnp.int32, (BLK, N_K), 1)
    eye = (rows == cols).astype(jnp.float32)
    p = eye + g_blk.astype(jnp.float32) + g2_blk.astype(jnp.float32)
    p_ref[...] = p.astype(jnp.bfloat16)


def _final_body(g3_blk_ref, p_full_ref, o_ref):
    i = pl.program_id(0)
    r = jnp.dot(g3_blk_ref[...], p_full_ref[...],
                preferred_element_type=jnp.float32)
    p_blk = p_full_ref[pl.ds(i * BLK, BLK), :].astype(jnp.float32)
    o_ref[...] = r + p_blk


def _row_blk(i):
    return (i, 0)


def _const_blk(i):
    return (0, 0)


def _dense_gstar(a_2d, Dx, idx_col):
    del idx_col
    bf16 = jnp.bfloat16
    f32 = jnp.float32
    dxb = Dx.astype(bf16)
    g = pl.pallas_call(
        _g_body,
        grid=(N_BLKS,),
        in_specs=[
            pl.BlockSpec((BLK, D), _row_blk),
            pl.BlockSpec((N_K, D), _const_blk),
            pl.BlockSpec((1, BLK), lambda i: (0, i)),
            pl.BlockSpec((1, N_K), _const_blk),
        ],
        out_specs=pl.BlockSpec((BLK, N_K), _row_blk),
        out_shape=jax.ShapeDtypeStruct((N_K, N_K), bf16),
    )(dxb, dxb, a_2d, a_2d)

    mm_specs = dict(
        grid=(N_BLKS,),
        in_specs=[
            pl.BlockSpec((BLK, N_K), _row_blk),
            pl.BlockSpec((N_K, N_K), _const_blk),
        ],
    )
    g2 = pl.pallas_call(
        _mm_body,
        out_specs=pl.BlockSpec((BLK, N_K), _row_blk),
        out_shape=jax.ShapeDtypeStruct((N_K, N_K), bf16),
        **mm_specs,
    )(g, g)
    g3, p = pl.pallas_call(
        _mm_p_body,
        out_specs=(
            pl.BlockSpec((BLK, N_K), _row_blk),
            pl.BlockSpec((BLK, N_K), _row_blk),
        ),
        out_shape=(
            jax.ShapeDtypeStruct((N_K, N_K), bf16),
            jax.ShapeDtypeStruct((N_K, N_K), bf16),
        ),
        **mm_specs,
    )(g, g2)
    g_star = pl.pallas_call(
        _final_body,
        out_specs=pl.BlockSpec((BLK, N_K), _row_blk),
        out_shape=jax.ShapeDtypeStruct((N_K, N_K), f32),
        **mm_specs,
    )(g3, p)
    return g_star


def _compressed_gstar(a_2d, Dx, idx_col):
    bf16 = jnp.bfloat16
    f32 = jnp.float32
    a_col = a_2d.reshape(N_K, 1)

    gcn, vb = pl.pallas_call(
        _mega1_body,
        out_shape=(
            jax.ShapeDtypeStruct((B, N_K), bf16),
            jax.ShapeDtypeStruct((B, N_K), bf16),
        ),
    )(Dx, a_2d, a_col, idx_col)

    g_star = pl.pallas_call(
        _mega2_body,
        grid=(N_BLKS,),
        in_specs=[
            pl.BlockSpec((B, BLK), lambda i: (0, i)),
            pl.BlockSpec((B, N_K), _const_blk),
        ],
        out_specs=pl.BlockSpec((BLK, N_K), _row_blk),
        out_shape=jax.ShapeDtypeStruct((N_K, N_K), f32),
    )(gcn, vb)
    return g_star


def kernel(x_t, E, Dx, Dy):
    del Dy
    f32 = jnp.float32

    y_t, a_2d = pl.pallas_call(
        _prep_body,
        out_shape=(
            jax.ShapeDtypeStruct((4, N_K), f32),
            jax.ShapeDtypeStruct((1, N_K), f32),
        ),
    )(x_t, E)

    a_1d = a_2d.reshape(N_K)
    active = a_1d > 0.0
    csum = jnp.cumsum(active.astype(jnp.int32))
    m = csum[N_K - 1]
    ks = jnp.arange(1, B + 1, dtype=jnp.int32)
    idx = jnp.searchsorted(csum, ks).astype(jnp.int32)
    valid = jnp.arange(B, dtype=jnp.int32) < m
    idx = jnp.where(valid, idx, -1)
    idx_col = idx.reshape(B, 1)

    g_star = lax.cond(
        m <= B,
        lambda ops: _compressed_gstar(*ops),
        lambda ops: _dense_gstar(*ops),
        (a_2d, Dx, idx_col),
    )
    return (y_t, g_star)


# speedup vs baseline: 3.1440x; 3.1440x over previous
import jax
import jax.numpy as jnp
from jax import lax
from jax.experimental import pallas as pl
from jax.experimental.pallas import tpu as pltpu

N_K = 2048
D = 1024
THRESHOLD = 0.1
BLK = 256
N_BLKS = N_K // BLK
B = 1024


def _prep_body(x_ref, e_ref, y_ref, a_ref, csum_ref, m_ref):
    v = jnp.dot(x_ref[...], e_ref[...], preferred_element_type=jnp.float32)
    mu = jnp.mean(v, axis=-1, keepdims=True)
    var = jnp.mean((v - mu) ** 2, axis=-1, keepdims=True)
    ln = (v - mu) * jax.lax.rsqrt(var + 1e-5)
    y = jnp.maximum(ln, 0.0)
    y_ref[...] = y
    x0 = y[0:1, :]
    a = jnp.where(x0 > THRESHOLD, x0, 0.0)
    a_ref[...] = a

    s = (a > 0.0).astype(jnp.float32)
    sh = 1
    while sh < N_K:
        s = s + jnp.concatenate(
            [jnp.zeros((1, sh), jnp.float32), s[:, : N_K - sh]], axis=1)
        sh *= 2
    csum_ref[...] = s
    m_ref[...] = jnp.zeros((1, 128), jnp.float32) + s[:, N_K - 1: N_K]


def _closure_body(dx_ref, a_row_ref, a_col_ref, csum_ref, o_ref, gcn_s, vb_s):
    i = pl.program_id(0)

    @pl.when(i == 0)
    def _():
        dx = dx_ref[...].astype(jnp.bfloat16)
        a_row = a_row_ref[...]
        csum = csum_ref[...]

        ks_col = jax.lax.broadcasted_iota(jnp.int32, (B, N_K), 0)
        csum_i = csum.astype(jnp.int32)
        sel = (csum_i - 1 == ks_col) & (a_row > 0.0)
        cb = sel.astype(jnp.bfloat16)

        dxc = jnp.dot(cb, dx, preferred_element_type=jnp.float32) \
            .astype(jnp.bfloat16)
        ac_col = jnp.dot(cb, a_col_ref[...].astype(jnp.bfloat16),
                         preferred_element_type=jnp.float32)
        ac_scaled = ac_col * (1.0 / N_K)

        gate = jax.nn.sigmoid(jax.lax.dot_general(
            dxc, dx, (((1,), (1,)), ((), ())),
            preferred_element_type=jnp.float32))
        gcn_f = jnp.where(sel, 0.0, gate * (ac_scaled * a_row))
        gcn = gcn_f.astype(jnp.bfloat16)
        gcn_s[...] = gcn

        gate_c = jax.nn.sigmoid(jax.lax.dot_general(
            dxc, dxc, (((1,), (1,)), ((), ())),
            preferred_element_type=jnp.float32))
        ii = jax.lax.broadcasted_iota(jnp.int32, (B, B), 0)
        jj = jax.lax.broadcasted_iota(jnp.int32, (B, B), 1)
        eq = ii == jj
        ac_row = ac_scaled.reshape(1, B)
        gc_f = jnp.where(eq, 0.0, gate_c * (ac_col * ac_row))
        gc = gc_f.astype(jnp.bfloat16)

        g2_f = jnp.dot(gc, gc, preferred_element_type=jnp.float32)
        g2 = g2_f.astype(jnp.bfloat16)
        mm = jnp.dot(gc, g2, preferred_element_type=jnp.float32)
        t3 = (jnp.where(eq, 1.0, 0.0) + gc_f + g2_f + mm).astype(jnp.bfloat16)

        v = jnp.dot(t3, gcn, preferred_element_type=jnp.float32)
        vb_s[...] = (v + sel.astype(jnp.float32)).astype(jnp.bfloat16)

    gcn_blk = gcn_s[:, pl.ds(i * BLK, BLK)]
    f = jax.lax.dot_general(
        gcn_blk, vb_s[...], (((0,), (0,)), ((), ())),
        preferred_element_type=jnp.float32)
    rows = jax.lax.broadcasted_iota(jnp.int32, (BLK, N_K), 0) + i * BLK
    cols = jax.lax.broadcasted_iota(jnp.int32, (BLK, N_K), 1)
    o_ref[...] = f + (rows == cols).astype(jnp.float32)


def _g_body(dx_blk_ref, dx_full_ref, a_blk_ref, a_full_ref, g_ref):
    i = pl.program_id(0)
    dots = jax.lax.dot_general(
        dx_blk_ref[...], dx_full_ref[...], (((1,), (1,)), ((), ())),
        preferred_element_type=jnp.float32)
    gate = jax.nn.sigmoid(dots)
    a_col = a_blk_ref[...].reshape(BLK, 1)
    a_row = a_full_ref[...]
    rows = jax.lax.broadcasted_iota(jnp.int32, (BLK, N_K), 0) + i * BLK
    cols = jax.lax.broadcasted_iota(jnp.int32, (BLK, N_K), 1)
    offdiag = (rows != cols).astype(jnp.float32)
    g = gate * (a_col * a_row) * offdiag * (1.0 / N_K)
    g_ref[...] = g.astype(jnp.bfloat16)


def _mm_body(a_blk_ref, b_full_ref, o_ref):
    o_ref[...] = jnp.dot(
        a_blk_ref[...], b_full_ref[...], preferred_element_type=jnp.float32
    ).astype(jnp.bfloat16)


def _mm_p_body(g_blk_ref, g2_full_ref, g3_ref, p_ref):
    i = pl.program_id(0)
    g_blk = g_blk_ref[...]
    g2_full = g2_full_ref[...]
    g3_ref[...] = jnp.dot(
        g_blk, g2_full, preferred_element_type=jnp.float32
    ).astype(jnp.bfloat16)
    g2_blk = g2_full_ref[pl.ds(i * BLK, BLK), :]
    rows = jax.lax.broadcasted_iota(jnp.int32, (BLK, N_K), 0) + i * BLK
    cols = jax.lax.broadcasted_iota(jnp.int32, (BLK, N_K), 1)
    eye = (rows == cols).astype(jnp.float32)
    p = eye + g_blk.astype(jnp.float32) + g2_blk.astype(jnp.float32)
    p_ref[...] = p.astype(jnp.bfloat16)


def _final_body(g3_blk_ref, p_full_ref, o_ref):
    i = pl.program_id(0)
    r = jnp.dot(g3_blk_ref[...], p_full_ref[...],
                preferred_element_type=jnp.float32)
    p_blk = p_full_ref[pl.ds(i * BLK, BLK), :].astype(jnp.float32)
    o_ref[...] = r + p_blk


def _row_blk(i):
    return (i, 0)


def _const_blk(i):
    return (0, 0)


def _dense_gstar(a_2d, Dx, csum):
    del csum
    bf16 = jnp.bfloat16
    f32 = jnp.float32
    dxb = Dx.astype(bf16)
    g = pl.pallas_call(
        _g_body,
        grid=(N_BLKS,),
        in_specs=[
            pl.BlockSpec((BLK, D), _row_blk),
            pl.BlockSpec((N_K, D), _const_blk),
            pl.BlockSpec((1, BLK), lambda i: (0, i)),
            pl.BlockSpec((1, N_K), _const_blk),
        ],
        out_specs=pl.BlockSpec((BLK, N_K), _row_blk),
        out_shape=jax.ShapeDtypeStruct((N_K, N_K), bf16),
    )(dxb, dxb, a_2d, a_2d)

    mm_specs = dict(
        grid=(N_BLKS,),
        in_specs=[
            pl.BlockSpec((BLK, N_K), _row_blk),
            pl.BlockSpec((N_K, N_K), _const_blk),
        ],
    )
    g2 = pl.pallas_call(
        _mm_body,
        out_specs=pl.BlockSpec((BLK, N_K), _row_blk),
        out_shape=jax.ShapeDtypeStruct((N_K, N_K), bf16),
        **mm_specs,
    )(g, g)
    g3, p = pl.pallas_call(
        _mm_p_body,
        out_specs=(
            pl.BlockSpec((BLK, N_K), _row_blk),
            pl.BlockSpec((BLK, N_K), _row_blk),
        ),
        out_shape=(
            jax.ShapeDtypeStruct((N_K, N_K), bf16),
            jax.ShapeDtypeStruct((N_K, N_K), bf16),
        ),
        **mm_specs,
    )(g, g2)
    g_star = pl.pallas_call(
        _final_body,
        out_specs=pl.BlockSpec((BLK, N_K), _row_blk),
        out_shape=jax.ShapeDtypeStruct((N_K, N_K), f32),
        **mm_specs,
    )(g3, p)
    return g_star


def _compressed_gstar(a_2d, Dx, csum):
    bf16 = jnp.bfloat16
    f32 = jnp.float32
    a_col = a_2d.reshape(N_K, 1)
    return pl.pallas_call(
        _closure_body,
        grid=(N_BLKS,),
        in_specs=[
            pl.BlockSpec((N_K, D), _const_blk),
            pl.BlockSpec((1, N_K), _const_blk),
            pl.BlockSpec((N_K, 1), _const_blk),
            pl.BlockSpec((1, N_K), _const_blk),
        ],
        out_specs=pl.BlockSpec((BLK, N_K), _row_blk),
        out_shape=jax.ShapeDtypeStruct((N_K, N_K), f32),
        scratch_shapes=[
            pltpu.VMEM((B, N_K), bf16),
            pltpu.VMEM((B, N_K), bf16),
        ],
    )(Dx, a_2d, a_col, csum)


def kernel(x_t, E, Dx, Dy):
    del Dy
    f32 = jnp.float32

    y_t, a_2d, csum, m_arr = pl.pallas_call(
        _prep_body,
        out_shape=(
            jax.ShapeDtypeStruct((4, N_K), f32),
            jax.ShapeDtypeStruct((1, N_K), f32),
            jax.ShapeDtypeStruct((1, N_K), f32),
            jax.ShapeDtypeStruct((1, 128), f32),
        ),
    )(x_t, E)

    g_star = lax.cond(
        m_arr[0, 0] <= float(B),
        lambda ops: _compressed_gstar(*ops),
        lambda ops: _dense_gstar(*ops),
        (a_2d, Dx, csum),
    )
    return (y_t, g_star)
